# baseline (speedup 1.0000x reference)
import functools
import math

import jax
import jax.numpy as jnp
from jax.experimental import pallas as pl
from jax.experimental.pallas import tpu as pltpu

_T = 2048
_C = 768
_H = 12
_HD = _C // _H
_RB = 256
_CB = 256
_NCH = _T // _CB
_N_ITER = 24


def _qkv_body(x_ref, w_ref, b_ref, o_ref):
    o_ref[...] = jax.lax.dot_general(
        x_ref[...], w_ref[...], (((1,), (1,)), ((), ())),
        preferred_element_type=jnp.float32) + b_ref[...]


def _attn_body(ratio_ref, q_ref, k_ref, v_ref, o_ref, att_scr):
    h = pl.program_id(0)
    tb = pl.program_id(1)
    q = q_ref[0]
    scale = 1.0 / math.sqrt(_HD)
    rows = tb * _RB + jax.lax.broadcasted_iota(jnp.int32, (_RB, 1), 0)
    big = jnp.float32(3e38)
    nch = tb + 1

    def pass0(c, carry):
        hi_c, lo_c = carry
        kk = k_ref[0, pl.ds(c * _CB, _CB), :]
        a = jax.lax.dot_general(
            q, kk, (((1,), (1,)), ((), ())),
            preferred_element_type=jnp.float32) * scale
        colc = c * _CB + jax.lax.broadcasted_iota(jnp.int32, (_RB, _CB), 1)
        validc = colc <= rows
        a_m = jnp.where(validc, a, -big)
        att_scr[:, pl.ds(c * _CB, _CB)] = a_m
        hi_c = jnp.maximum(hi_c, jnp.max(a_m, axis=1, keepdims=True))
        lo_c = jnp.minimum(
            lo_c, jnp.min(jnp.where(validc, a, big), axis=1, keepdims=True))
        return hi_c, lo_c

    mrow, lo = jax.lax.fori_loop(
        0, nch, pass0,
        (jnp.full((_RB, 1), -big, jnp.float32),
         jnp.full((_RB, 1), big, jnp.float32)))

    r = ratio_ref[h]
    sig = 1.0 / (1.0 + jnp.exp(-r))
    tlen = (rows + 1).astype(jnp.float32)
    kt = jnp.maximum(1, jnp.floor(tlen * sig).astype(jnp.int32))
    ktf = kt.astype(jnp.float32)

    def bisect(_, carry):
        lo_c, hi_c = carry
        mid = (lo_c + hi_c) * 0.5

        def cbody(c, acc):
            blk = att_scr[:, pl.ds(c * _CB, _CB)]
            return acc + jnp.sum((blk >= mid).astype(jnp.float32), axis=1,
                                 keepdims=True)

        cnt = jax.lax.fori_loop(0, nch, cbody,
                                jnp.zeros((_RB, 1), jnp.float32))
        ge = cnt >= ktf
        return jnp.where(ge, mid, lo_c), jnp.where(ge, hi_c, mid)

    lo, _ = jax.lax.fori_loop(0, _N_ITER, bisect, (lo, mrow))

    m = jnp.maximum(mrow, 0.0)
    em = jnp.exp(-m)

    def pass2(c, carry):
        acc_y, acc_d = carry
        blk = att_scr[:, pl.ds(c * _CB, _CB)]
        s = jnp.where(blk >= lo, blk, 0.0)
        p = jnp.exp(s - m)
        acc_d = acc_d + jnp.sum(p, axis=1, keepdims=True)
        vv = v_ref[0, pl.ds(c * _CB, _CB), :]
        acc_y = acc_y + jax.lax.dot_general(
            p, vv, (((1,), (0,)), ((), ())),
            preferred_element_type=jnp.float32)
        return acc_y, acc_d

    acc_y, acc_d = jax.lax.fori_loop(
        0, nch, pass2,
        (jnp.zeros((_RB, _HD), jnp.float32),
         jnp.zeros((_RB, 1), jnp.float32)))

    def tailsum(c, acc):
        vv = v_ref[0, pl.ds(c * _CB, _CB), :]
        return acc + jnp.sum(vv, axis=0, keepdims=True)

    vtail = jax.lax.fori_loop(nch, _NCH, tailsum,
                              jnp.zeros((1, _HD), jnp.float32))
    ntail = (_T - nch * _CB).astype(jnp.float32)
    y = acc_y + em * vtail
    d = acc_d + em * ntail
    o_ref[0] = y / d


def _proj_body(y_ref, w_ref, b_ref, o_ref):
    o_ref[...] = jax.lax.dot_general(
        y_ref[...], w_ref[...], (((1,), (1,)), ((), ())),
        preferred_element_type=jnp.float32) + b_ref[...]


@jax.jit
def kernel(x, W_attn, b_attn, W_proj, b_proj, sparsity_ratios):
    B, T, C = x.shape
    H = sparsity_ratios.shape[0]
    hd = C // H
    x2 = x.reshape(T, C)

    qkv = pl.pallas_call(
        _qkv_body,
        grid=(T // _RB,),
        in_specs=[
            pl.BlockSpec((_RB, C), lambda i: (i, 0)),
            pl.BlockSpec((3 * C, C), lambda i: (0, 0)),
            pl.BlockSpec((1, 3 * C), lambda i: (0, 0)),
        ],
        out_specs=pl.BlockSpec((_RB, 3 * C), lambda i: (i, 0)),
        out_shape=jax.ShapeDtypeStruct((T, 3 * C), jnp.float32),
    )(x2, W_attn, b_attn.reshape(1, 3 * C))

    q = qkv[:, :C].reshape(T, H, hd).transpose(1, 0, 2)
    k = qkv[:, C:2 * C].reshape(T, H, hd).transpose(1, 0, 2)
    v = qkv[:, 2 * C:].reshape(T, H, hd).transpose(1, 0, 2)

    grid_spec = pltpu.PrefetchScalarGridSpec(
        num_scalar_prefetch=1,
        grid=(H, T // _RB),
        in_specs=[
            pl.BlockSpec((1, _RB, hd), lambda h, t, *_: (h, t, 0)),
            pl.BlockSpec((1, T, hd), lambda h, t, *_: (h, 0, 0)),
            pl.BlockSpec((1, T, hd), lambda h, t, *_: (h, 0, 0)),
        ],
        out_specs=pl.BlockSpec((1, _RB, hd), lambda h, t, *_: (h, t, 0)),
        scratch_shapes=[pltpu.VMEM((_RB, _T), jnp.float32)],
    )
    y = pl.pallas_call(
        _attn_body,
        grid_spec=grid_spec,
        out_shape=jax.ShapeDtypeStruct((H, T, hd), jnp.float32),
        compiler_params=pltpu.CompilerParams(
            dimension_semantics=("arbitrary", "arbitrary")),
    )(sparsity_ratios, q, k, v)

    y2 = y.transpose(1, 0, 2).reshape(T, C)
    out = pl.pallas_call(
        _proj_body,
        grid=(T // _RB,),
        in_specs=[
            pl.BlockSpec((_RB, C), lambda i: (i, 0)),
            pl.BlockSpec((C, C), lambda i: (0, 0)),
            pl.BlockSpec((1, C), lambda i: (0, 0)),
        ],
        out_specs=pl.BlockSpec((_RB, C), lambda i: (i, 0)),
        out_shape=jax.ShapeDtypeStruct((T, C), jnp.float32),
    )(y2, W_proj, b_proj.reshape(1, C))
    return out.reshape(B, T, C)


# speedup vs baseline: 1.3396x; 1.3396x over previous
import functools
import math

import jax
import jax.numpy as jnp
from jax.experimental import pallas as pl
from jax.experimental.pallas import tpu as pltpu

_T = 2048
_C = 768
_H = 12
_HD = _C // _H
_RB = 256
_CB = 512
_NCH = _T // _CB
_N_ITER = 26


def _qkv_body(x_ref, w_ref, b_ref, o_ref):
    o_ref[...] = jax.lax.dot_general(
        x_ref[...], w_ref[...], (((1,), (1,)), ((), ())),
        preferred_element_type=jnp.float32) + b_ref[...]


def _attn_body(ratio_ref, q_ref, k_ref, v_ref, o_ref, att_scr):
    h = pl.program_id(0)
    tb = pl.program_id(1)
    q = q_ref[0]
    scale = 1.0 / math.sqrt(_HD)
    rows = tb * _RB + jax.lax.broadcasted_iota(jnp.int32, (_RB, 1), 0)
    big = jnp.float32(3e38)
    nch = tb * _RB // _CB + 1

    def pass0(c, carry):
        hi_c, lo_c = carry
        kk = k_ref[0, pl.ds(c * _CB, _CB), :]
        a = jax.lax.dot_general(
            q, kk, (((1,), (1,)), ((), ())),
            preferred_element_type=jnp.float32) * scale
        colc = c * _CB + jax.lax.broadcasted_iota(jnp.int32, (_RB, _CB), 1)
        validc = colc <= rows
        a_m = jnp.where(validc, a, -big)
        att_scr[c] = a_m
        hi_c = jnp.maximum(hi_c, jnp.max(a_m, axis=1, keepdims=True))
        lo_c = jnp.minimum(
            lo_c, jnp.min(jnp.where(validc, a, big), axis=1, keepdims=True))
        return hi_c, lo_c

    mrow, lo = jax.lax.fori_loop(
        0, nch, pass0,
        (jnp.full((_RB, 1), -big, jnp.float32),
         jnp.full((_RB, 1), big, jnp.float32)))

    r = ratio_ref[h]
    sig = 1.0 / (1.0 + jnp.exp(-r))
    tlen = (rows + 1).astype(jnp.float32)
    kt = jnp.maximum(1, jnp.floor(tlen * sig).astype(jnp.int32))
    ktf = kt.astype(jnp.float32)

    def bisect(_, carry):
        lo_c, hi_c = carry
        mid = (lo_c + hi_c) * 0.5

        def cbody(c, acc):
            blk = att_scr[c]
            return acc + jnp.sum((blk >= mid).astype(jnp.float32), axis=1,
                                 keepdims=True)

        cnt = jax.lax.fori_loop(0, nch, cbody,
                                jnp.zeros((_RB, 1), jnp.float32))
        ge = cnt >= ktf
        return jnp.where(ge, mid, lo_c), jnp.where(ge, hi_c, mid)

    lo, _ = jax.lax.fori_loop(0, _N_ITER, bisect, (lo, mrow))

    m = jnp.maximum(mrow, 0.0)
    em = jnp.exp(-m)

    def pass2(c, carry):
        acc_y, acc_d = carry
        blk = att_scr[c]
        s = jnp.where(blk >= lo, blk, 0.0)
        p = jnp.exp(s - m)
        acc_d = acc_d + jnp.sum(p, axis=1, keepdims=True)
        vv = v_ref[0, pl.ds(c * _CB, _CB), :]
        acc_y = acc_y + jax.lax.dot_general(
            p, vv, (((1,), (0,)), ((), ())),
            preferred_element_type=jnp.float32)
        return acc_y, acc_d

    acc_y, acc_d = jax.lax.fori_loop(
        0, nch, pass2,
        (jnp.zeros((_RB, _HD), jnp.float32),
         jnp.zeros((_RB, 1), jnp.float32)))

    def tailsum(c, acc):
        vv = v_ref[0, pl.ds(c * _CB, _CB), :]
        return acc + jnp.sum(vv, axis=0, keepdims=True)

    vtail = jax.lax.fori_loop(nch, _NCH, tailsum,
                              jnp.zeros((1, _HD), jnp.float32))
    ntail = (_T - nch * _CB).astype(jnp.float32)
    y = acc_y + em * vtail
    d = acc_d + em * ntail
    o_ref[0] = y / d


def _proj_body(y_ref, w_ref, b_ref, o_ref):
    o_ref[...] = jax.lax.dot_general(
        y_ref[...], w_ref[...], (((1,), (1,)), ((), ())),
        preferred_element_type=jnp.float32) + b_ref[...]


@jax.jit
def kernel(x, W_attn, b_attn, W_proj, b_proj, sparsity_ratios):
    B, T, C = x.shape
    H = sparsity_ratios.shape[0]
    hd = C // H
    x2 = x.reshape(T, C)

    qkv = pl.pallas_call(
        _qkv_body,
        grid=(T // _RB,),
        in_specs=[
            pl.BlockSpec((_RB, C), lambda i: (i, 0)),
            pl.BlockSpec((3 * C, C), lambda i: (0, 0)),
            pl.BlockSpec((1, 3 * C), lambda i: (0, 0)),
        ],
        out_specs=pl.BlockSpec((_RB, 3 * C), lambda i: (i, 0)),
        out_shape=jax.ShapeDtypeStruct((T, 3 * C), jnp.float32),
    )(x2, W_attn, b_attn.reshape(1, 3 * C))

    q = qkv[:, :C].reshape(T, H, hd).transpose(1, 0, 2)
    k = qkv[:, C:2 * C].reshape(T, H, hd).transpose(1, 0, 2)
    v = qkv[:, 2 * C:].reshape(T, H, hd).transpose(1, 0, 2)

    grid_spec = pltpu.PrefetchScalarGridSpec(
        num_scalar_prefetch=1,
        grid=(H, T // _RB),
        in_specs=[
            pl.BlockSpec((1, _RB, hd), lambda h, t, *_: (h, t, 0)),
            pl.BlockSpec((1, T, hd), lambda h, t, *_: (h, 0, 0)),
            pl.BlockSpec((1, T, hd), lambda h, t, *_: (h, 0, 0)),
        ],
        out_specs=pl.BlockSpec((1, _RB, hd), lambda h, t, *_: (h, t, 0)),
        scratch_shapes=[pltpu.VMEM((_NCH, _RB, _CB), jnp.float32)],
    )
    y = pl.pallas_call(
        _attn_body,
        grid_spec=grid_spec,
        out_shape=jax.ShapeDtypeStruct((H, T, hd), jnp.float32),
        compiler_params=pltpu.CompilerParams(
            dimension_semantics=("arbitrary", "arbitrary")),
    )(sparsity_ratios, q, k, v)

    y2 = y.transpose(1, 0, 2).reshape(T, C)
    out = pl.pallas_call(
        _proj_body,
        grid=(T // _RB,),
        in_specs=[
            pl.BlockSpec((_RB, C), lambda i: (i, 0)),
            pl.BlockSpec((C, C), lambda i: (0, 0)),
            pl.BlockSpec((1, C), lambda i: (0, 0)),
        ],
        out_specs=pl.BlockSpec((_RB, C), lambda i: (i, 0)),
        out_shape=jax.ShapeDtypeStruct((T, C), jnp.float32),
    )(y2, W_proj, b_proj.reshape(1, C))
    return out.reshape(B, T, C)


# speedup vs baseline: 1.9571x; 1.4609x over previous
import functools
import math

import jax
import jax.numpy as jnp
from jax.experimental import pallas as pl
from jax.experimental.pallas import tpu as pltpu

_T = 2048
_C = 768
_H = 12
_HD = _C // _H
_RB = 256
_GR = 512
_N_ITER = 28


def _qkv_body(x_ref, w_ref, b_ref, o_ref):
    o_ref[...] = jax.lax.dot_general(
        x_ref[...], w_ref[...], (((1,), (1,)), ((), ())),
        preferred_element_type=jnp.float32) + b_ref[...]


def _attn_group_body(W, ROFF, ratio_ref, q_ref, k_ref, v_ref, o_ref):
    h = pl.program_id(0)
    tb = pl.program_id(1)
    q = q_ref[0]
    k = k_ref[0]
    scale = 1.0 / math.sqrt(_HD)
    att = jax.lax.dot_general(
        q, k, (((1,), (1,)), ((), ())),
        preferred_element_type=jnp.float32) * scale

    rows = ROFF + tb * _RB + jax.lax.broadcasted_iota(jnp.int32, (_RB, 1), 0)
    cols = jax.lax.broadcasted_iota(jnp.int32, (_RB, W), 1)
    valid = cols <= rows

    big = jnp.float32(3e38)
    att_m = jnp.where(valid, att, -big)
    mrow = jnp.max(att_m, axis=1, keepdims=True)
    lo = jnp.min(jnp.where(valid, att, big), axis=1, keepdims=True)

    r = ratio_ref[h]
    sig = 1.0 / (1.0 + jnp.exp(-r))
    tlen = (rows + 1).astype(jnp.float32)
    kt = jnp.maximum(1, jnp.floor(tlen * sig).astype(jnp.int32))
    ktf = kt.astype(jnp.float32)

    def bisect(_, carry):
        lo_c, hi_c = carry
        mid = (lo_c + hi_c) * 0.5
        cnt = jnp.sum((att_m >= mid).astype(jnp.float32), axis=1,
                      keepdims=True)
        ge = cnt >= ktf
        return jnp.where(ge, mid, lo_c), jnp.where(ge, hi_c, mid)

    lo, _ = jax.lax.fori_loop(0, _N_ITER, bisect, (lo, mrow))

    m = jnp.maximum(mrow, 0.0)
    s = jnp.where(att_m >= lo, att_m, 0.0)
    p = jnp.exp(s - m)
    num = jax.lax.dot_general(
        p, v_ref[0, :W, :], (((1,), (0,)), ((), ())),
        preferred_element_type=jnp.float32)
    den = jnp.sum(p, axis=1, keepdims=True)
    if W < _T:
        em = jnp.exp(-m)
        vtail = jnp.sum(v_ref[0, W:, :], axis=0, keepdims=True)
        num = num + em * vtail
        den = den + em * jnp.float32(_T - W)
    o_ref[0] = num / den


def _proj_body(y_ref, w_ref, b_ref, o_ref):
    o_ref[...] = jax.lax.dot_general(
        y_ref[...], w_ref[...], (((1,), (1,)), ((), ())),
        preferred_element_type=jnp.float32) + b_ref[...]


@jax.jit
def kernel(x, W_attn, b_attn, W_proj, b_proj, sparsity_ratios):
    B, T, C = x.shape
    H = sparsity_ratios.shape[0]
    hd = C // H
    x2 = x.reshape(T, C)

    qkv = pl.pallas_call(
        _qkv_body,
        grid=(T // _RB,),
        in_specs=[
            pl.BlockSpec((_RB, C), lambda i: (i, 0)),
            pl.BlockSpec((3 * C, C), lambda i: (0, 0)),
            pl.BlockSpec((1, 3 * C), lambda i: (0, 0)),
        ],
        out_specs=pl.BlockSpec((_RB, 3 * C), lambda i: (i, 0)),
        out_shape=jax.ShapeDtypeStruct((T, 3 * C), jnp.float32),
    )(x2, W_attn, b_attn.reshape(1, 3 * C))

    q = qkv[:, :C].reshape(T, H, hd).transpose(1, 0, 2)
    k = qkv[:, C:2 * C].reshape(T, H, hd).transpose(1, 0, 2)
    v = qkv[:, 2 * C:].reshape(T, H, hd).transpose(1, 0, 2)

    y_groups = []
    for g in range(T // _GR):
        roff = g * _GR
        w_g = roff + _GR
        grid_spec = pltpu.PrefetchScalarGridSpec(
            num_scalar_prefetch=1,
            grid=(H, _GR // _RB),
            in_specs=[
                pl.BlockSpec((1, _RB, hd),
                             lambda h, t, *_, _o=roff // _RB: (h, t + _o, 0)),
                pl.BlockSpec((1, w_g, hd), lambda h, t, *_: (h, 0, 0)),
                pl.BlockSpec((1, T, hd), lambda h, t, *_: (h, 0, 0)),
            ],
            out_specs=pl.BlockSpec((1, _RB, hd), lambda h, t, *_: (h, t, 0)),
        )
        y_g = pl.pallas_call(
            functools.partial(_attn_group_body, w_g, roff),
            grid_spec=grid_spec,
            out_shape=jax.ShapeDtypeStruct((H, _GR, hd), jnp.float32),
            compiler_params=pltpu.CompilerParams(
                dimension_semantics=("arbitrary", "arbitrary")),
        )(sparsity_ratios, q, k, v)
        y_groups.append(y_g)
    y = jnp.concatenate(y_groups, axis=1)

    y2 = y.transpose(1, 0, 2).reshape(T, C)
    out = pl.pallas_call(
        _proj_body,
        grid=(T // _RB,),
        in_specs=[
            pl.BlockSpec((_RB, C), lambda i: (i, 0)),
            pl.BlockSpec((C, C), lambda i: (0, 0)),
            pl.BlockSpec((1, C), lambda i: (0, 0)),
        ],
        out_specs=pl.BlockSpec((_RB, C), lambda i: (i, 0)),
        out_shape=jax.ShapeDtypeStruct((T, C), jnp.float32),
    )(y2, W_proj, b_proj.reshape(1, C))
    return out.reshape(B, T, C)


# speedup vs baseline: 2.8524x; 1.4575x over previous
import functools
import math

import jax
import jax.numpy as jnp
from jax.experimental import pallas as pl
from jax.experimental.pallas import tpu as pltpu

_T = 2048
_C = 768
_H = 12
_HD = _C // _H
_RB = 256
_GR = 512
_N_ITER = 16


def _qkv_body(x_ref, w_ref, b_ref, o_ref):
    o_ref[...] = jax.lax.dot_general(
        x_ref[...], w_ref[...], (((1,), (1,)), ((), ())),
        preferred_element_type=jnp.float32) + b_ref[...]


def _attn_group_body(W, ROFF, ratio_ref, q_ref, k_ref, v_ref, o_ref):
    h = pl.program_id(0)
    tb = pl.program_id(1)
    q = q_ref[0]
    k = k_ref[0]
    scale = 1.0 / math.sqrt(_HD)
    att = jax.lax.dot_general(
        q, k, (((1,), (1,)), ((), ())),
        preferred_element_type=jnp.float32) * scale

    rows = ROFF + tb * _RB + jax.lax.broadcasted_iota(jnp.int32, (_RB, 1), 0)
    cols = jax.lax.broadcasted_iota(jnp.int32, (_RB, W), 1)
    valid = cols <= rows

    big = jnp.float32(3e38)
    att_m = jnp.where(valid, att, -big)
    mrow = jnp.max(att_m, axis=1, keepdims=True)
    lo = jnp.min(jnp.where(valid, att, big), axis=1, keepdims=True)

    r = ratio_ref[h]
    sig = 1.0 / (1.0 + jnp.exp(-r))
    tlen = (rows + 1).astype(jnp.float32)
    kt = jnp.maximum(1, jnp.floor(tlen * sig).astype(jnp.int32))
    ktf = kt.astype(jnp.float32)

    def bisect(_, carry):
        lo_c, hi_c = carry
        mid = (lo_c + hi_c) * 0.5
        cnt = jnp.sum((att_m >= mid).astype(jnp.float32), axis=1,
                      keepdims=True)
        ge = cnt >= ktf
        return jnp.where(ge, mid, lo_c), jnp.where(ge, hi_c, mid)

    lo, _ = jax.lax.fori_loop(0, _N_ITER, bisect, (lo, mrow))

    m = jnp.maximum(mrow, 0.0)
    s = jnp.where(att_m >= lo, att_m, 0.0)
    p = jnp.exp(s - m)
    num = jax.lax.dot_general(
        p, v_ref[0, :W, :], (((1,), (0,)), ((), ())),
        preferred_element_type=jnp.float32)
    den = jnp.sum(p, axis=1, keepdims=True)
    if W < _T:
        em = jnp.exp(-m)
        vtail = jnp.sum(v_ref[0, W:, :], axis=0, keepdims=True)
        num = num + em * vtail
        den = den + em * jnp.float32(_T - W)
    o_ref[0] = num / den


def _proj_body(y_ref, w_ref, b_ref, o_ref):
    o_ref[...] = jax.lax.dot_general(
        y_ref[...], w_ref[...], (((1,), (1,)), ((), ())),
        preferred_element_type=jnp.float32) + b_ref[...]


@jax.jit
def kernel(x, W_attn, b_attn, W_proj, b_proj, sparsity_ratios):
    B, T, C = x.shape
    H = sparsity_ratios.shape[0]
    hd = C // H
    x2 = x.reshape(T, C)

    qkv = pl.pallas_call(
        _qkv_body,
        grid=(T // _RB,),
        in_specs=[
            pl.BlockSpec((_RB, C), lambda i: (i, 0)),
            pl.BlockSpec((3 * C, C), lambda i: (0, 0)),
            pl.BlockSpec((1, 3 * C), lambda i: (0, 0)),
        ],
        out_specs=pl.BlockSpec((_RB, 3 * C), lambda i: (i, 0)),
        out_shape=jax.ShapeDtypeStruct((T, 3 * C), jnp.float32),
    )(x2, W_attn, b_attn.reshape(1, 3 * C))

    q = qkv[:, :C].reshape(T, H, hd).transpose(1, 0, 2)
    k = qkv[:, C:2 * C].reshape(T, H, hd).transpose(1, 0, 2)
    v = qkv[:, 2 * C:].reshape(T, H, hd).transpose(1, 0, 2)

    y_groups = []
    for g in range(T // _GR):
        roff = g * _GR
        w_g = roff + _GR
        grid_spec = pltpu.PrefetchScalarGridSpec(
            num_scalar_prefetch=1,
            grid=(H, _GR // _RB),
            in_specs=[
                pl.BlockSpec((1, _RB, hd),
                             lambda h, t, *_, _o=roff // _RB: (h, t + _o, 0)),
                pl.BlockSpec((1, w_g, hd), lambda h, t, *_: (h, 0, 0)),
                pl.BlockSpec((1, T, hd), lambda h, t, *_: (h, 0, 0)),
            ],
            out_specs=pl.BlockSpec((1, _RB, hd), lambda h, t, *_: (h, t, 0)),
        )
        y_g = pl.pallas_call(
            functools.partial(_attn_group_body, w_g, roff),
            grid_spec=grid_spec,
            out_shape=jax.ShapeDtypeStruct((H, _GR, hd), jnp.float32),
            compiler_params=pltpu.CompilerParams(
                dimension_semantics=("arbitrary", "arbitrary")),
        )(sparsity_ratios, q, k, v)
        y_groups.append(y_g)
    y = jnp.concatenate(y_groups, axis=1)

    y2 = y.transpose(1, 0, 2).reshape(T, C)
    out = pl.pallas_call(
        _proj_body,
        grid=(T // _RB,),
        in_specs=[
            pl.BlockSpec((_RB, C), lambda i: (i, 0)),
            pl.BlockSpec((C, C), lambda i: (0, 0)),
            pl.BlockSpec((1, C), lambda i: (0, 0)),
        ],
        out_specs=pl.BlockSpec((_RB, C), lambda i: (i, 0)),
        out_shape=jax.ShapeDtypeStruct((T, C), jnp.float32),
    )(y2, W_proj, b_proj.reshape(1, C))
    return out.reshape(B, T, C)


# speedup vs baseline: 3.3363x; 1.1696x over previous
import functools
import math

import jax
import jax.numpy as jnp
from jax.experimental import pallas as pl
from jax.experimental.pallas import tpu as pltpu

_T = 2048
_C = 768
_H = 12
_HD = _C // _H
_RB = 256
_GR = 512
_N_ITER = 16


def _qkv_body(x_ref, w_ref, b_ref, o_ref):
    o_ref[...] = jax.lax.dot_general(
        x_ref[...], w_ref[...], (((1,), (1,)), ((), ())),
        preferred_element_type=jnp.float32) + b_ref[...]


def _attn_group_body(W, ROFF, ratio_ref, q_ref, k_ref, v_ref, o_ref):
    hp = pl.program_id(0)
    tb = pl.program_id(1)
    scale = 1.0 / math.sqrt(_HD)
    rows = ROFF + tb * _RB + jax.lax.broadcasted_iota(jnp.int32, (_RB, 1), 0)
    cols = jax.lax.broadcasted_iota(jnp.int32, (_RB, W), 1)
    valid = cols <= rows
    big = jnp.float32(3e38)
    tlen = (rows + 1).astype(jnp.float32)

    for sub in range(2):
        h = 2 * hp + sub
        q = q_ref[:, sub * _HD:(sub + 1) * _HD]
        k = k_ref[:W, sub * _HD:(sub + 1) * _HD]
        att = jax.lax.dot_general(
            q, k, (((1,), (1,)), ((), ())),
            preferred_element_type=jnp.float32) * scale

        att_m = jnp.where(valid, att, -big)
        mrow = jnp.max(att_m, axis=1, keepdims=True)
        lo = jnp.min(jnp.where(valid, att, big), axis=1, keepdims=True)

        r = ratio_ref[h]
        sig = 1.0 / (1.0 + jnp.exp(-r))
        kt = jnp.maximum(1, jnp.floor(tlen * sig).astype(jnp.int32))
        ktf = kt.astype(jnp.float32)

        def bisect(_, carry):
            lo_c, hi_c = carry
            mid = (lo_c + hi_c) * 0.5
            cnt = jnp.sum((att_m >= mid).astype(jnp.float32), axis=1,
                          keepdims=True)
            ge = cnt >= ktf
            return jnp.where(ge, mid, lo_c), jnp.where(ge, hi_c, mid)

        lo, _ = jax.lax.fori_loop(0, _N_ITER, bisect, (lo, mrow))

        m = jnp.maximum(mrow, 0.0)
        s = jnp.where(att_m >= lo, att_m, 0.0)
        p = jnp.exp(s - m)
        num = jax.lax.dot_general(
            p, v_ref[:W, sub * _HD:(sub + 1) * _HD], (((1,), (0,)), ((), ())),
            preferred_element_type=jnp.float32)
        den = jnp.sum(p, axis=1, keepdims=True)
        if W < _T:
            em = jnp.exp(-m)
            vtail = jnp.sum(v_ref[W:, sub * _HD:(sub + 1) * _HD], axis=0,
                            keepdims=True)
            num = num + em * vtail
            den = den + em * jnp.float32(_T - W)
        o_ref[:, sub * _HD:(sub + 1) * _HD] = num / den


def _proj_body(y_ref, w_ref, b_ref, o_ref):
    o_ref[...] = jax.lax.dot_general(
        y_ref[...], w_ref[...], (((1,), (1,)), ((), ())),
        preferred_element_type=jnp.float32) + b_ref[...]


@jax.jit
def kernel(x, W_attn, b_attn, W_proj, b_proj, sparsity_ratios):
    B, T, C = x.shape
    H = sparsity_ratios.shape[0]
    hd = C // H
    x2 = x.reshape(T, C)

    qkv = pl.pallas_call(
        _qkv_body,
        grid=(T // _RB,),
        in_specs=[
            pl.BlockSpec((_RB, C), lambda i: (i, 0)),
            pl.BlockSpec((3 * C, C), lambda i: (0, 0)),
            pl.BlockSpec((1, 3 * C), lambda i: (0, 0)),
        ],
        out_specs=pl.BlockSpec((_RB, 3 * C), lambda i: (i, 0)),
        out_shape=jax.ShapeDtypeStruct((T, 3 * C), jnp.float32),
    )(x2, W_attn, b_attn.reshape(1, 3 * C))

    hpairs = H // 2
    y_groups = []
    for g in range(T // _GR):
        roff = g * _GR
        w_g = roff + _GR
        grid_spec = pltpu.PrefetchScalarGridSpec(
            num_scalar_prefetch=1,
            grid=(hpairs, _GR // _RB),
            in_specs=[
                pl.BlockSpec((_RB, 128),
                             lambda h, t, *_, _o=roff // _RB: (t + _o, h)),
                pl.BlockSpec((T, 128), lambda h, t, *_: (0, hpairs + h)),
                pl.BlockSpec((T, 128), lambda h, t, *_: (0, 2 * hpairs + h)),
            ],
            out_specs=pl.BlockSpec((_RB, 128), lambda h, t, *_: (t, h)),
        )
        y_g = pl.pallas_call(
            functools.partial(_attn_group_body, w_g, roff),
            grid_spec=grid_spec,
            out_shape=jax.ShapeDtypeStruct((_GR, C), jnp.float32),
            compiler_params=pltpu.CompilerParams(
                dimension_semantics=("arbitrary", "arbitrary")),
        )(sparsity_ratios, qkv, qkv, qkv)
        y_groups.append(y_g)
    y2 = jnp.concatenate(y_groups, axis=0)
    out = pl.pallas_call(
        _proj_body,
        grid=(T // _RB,),
        in_specs=[
            pl.BlockSpec((_RB, C), lambda i: (i, 0)),
            pl.BlockSpec((C, C), lambda i: (0, 0)),
            pl.BlockSpec((1, C), lambda i: (0, 0)),
        ],
        out_specs=pl.BlockSpec((_RB, C), lambda i: (i, 0)),
        out_shape=jax.ShapeDtypeStruct((T, C), jnp.float32),
    )(y2, W_proj, b_proj.reshape(1, C))
    return out.reshape(B, T, C)


# speedup vs baseline: 4.0582x; 1.2164x over previous
import functools
import math

import jax
import jax.numpy as jnp
from jax.experimental import pallas as pl
from jax.experimental.pallas import tpu as pltpu

_T = 2048
_C = 768
_H = 12
_HD = _C // _H
_RB = 256
_GR = 512
_N_ITER = 12


def _qkv_body(x_ref, w_ref, b_ref, o_ref):
    o_ref[...] = jax.lax.dot_general(
        x_ref[...], w_ref[...], (((1,), (1,)), ((), ())),
        preferred_element_type=jnp.float32) + b_ref[...]


def _attn_group_body(W, ROFF, ratio_ref, q_ref, k_ref, v_ref, o_ref):
    hp = pl.program_id(0)
    tb = pl.program_id(1)
    scale = 1.0 / math.sqrt(_HD)
    rows = ROFF + tb * _RB + jax.lax.broadcasted_iota(jnp.int32, (_RB, 1), 0)
    cols = jax.lax.broadcasted_iota(jnp.int32, (_RB, W), 1)
    valid = cols <= rows
    big = jnp.float32(3e38)
    tlen = (rows + 1).astype(jnp.float32)

    for sub in range(2):
        h = 2 * hp + sub
        q = q_ref[:, sub * _HD:(sub + 1) * _HD]
        k = k_ref[:W, sub * _HD:(sub + 1) * _HD]
        att = jax.lax.dot_general(
            q, k, (((1,), (1,)), ((), ())),
            preferred_element_type=jnp.float32) * scale

        att_m = jnp.where(valid, att, -big)
        mrow = jnp.max(att_m, axis=1, keepdims=True)
        lo = jnp.min(jnp.where(valid, att, big), axis=1, keepdims=True)

        r = ratio_ref[h]
        sig = 1.0 / (1.0 + jnp.exp(-r))
        kt = jnp.maximum(1, jnp.floor(tlen * sig).astype(jnp.int32))
        ktf = kt.astype(jnp.float32)

        def bisect(_, carry):
            lo_c, hi_c = carry
            mid = (lo_c + hi_c) * 0.5
            cnt = jnp.sum((att_m >= mid).astype(jnp.float32), axis=1,
                          keepdims=True)
            ge = cnt >= ktf
            return jnp.where(ge, mid, lo_c), jnp.where(ge, hi_c, mid)

        lo, _ = jax.lax.fori_loop(0, _N_ITER, bisect, (lo, mrow))

        m = jnp.maximum(mrow, 0.0)
        s = jnp.where(att_m >= lo, att_m, 0.0)
        p = jnp.exp(s - m)
        num = jax.lax.dot_general(
            p, v_ref[:W, sub * _HD:(sub + 1) * _HD], (((1,), (0,)), ((), ())),
            preferred_element_type=jnp.float32)
        den = jnp.sum(p, axis=1, keepdims=True)
        if W < _T:
            em = jnp.exp(-m)
            vtail = jnp.sum(v_ref[W:, sub * _HD:(sub + 1) * _HD], axis=0,
                            keepdims=True)
            num = num + em * vtail
            den = den + em * jnp.float32(_T - W)
        o_ref[:, sub * _HD:(sub + 1) * _HD] = num / den


def _proj_body(y_ref, w_ref, b_ref, o_ref):
    o_ref[...] = jax.lax.dot_general(
        y_ref[...], w_ref[...], (((1,), (1,)), ((), ())),
        preferred_element_type=jnp.float32) + b_ref[...]


@jax.jit
def kernel(x, W_attn, b_attn, W_proj, b_proj, sparsity_ratios):
    B, T, C = x.shape
    H = sparsity_ratios.shape[0]
    hd = C // H
    x2 = x.reshape(T, C)

    qkv = pl.pallas_call(
        _qkv_body,
        grid=(T // _RB,),
        in_specs=[
            pl.BlockSpec((_RB, C), lambda i: (i, 0)),
            pl.BlockSpec((3 * C, C), lambda i: (0, 0)),
            pl.BlockSpec((1, 3 * C), lambda i: (0, 0)),
        ],
        out_specs=pl.BlockSpec((_RB, 3 * C), lambda i: (i, 0)),
        out_shape=jax.ShapeDtypeStruct((T, 3 * C), jnp.float32),
    )(x2, W_attn, b_attn.reshape(1, 3 * C))

    hpairs = H // 2
    y_groups = []
    for g in range(T // _GR):
        roff = g * _GR
        w_g = roff + _GR
        grid_spec = pltpu.PrefetchScalarGridSpec(
            num_scalar_prefetch=1,
            grid=(hpairs, _GR // _RB),
            in_specs=[
                pl.BlockSpec((_RB, 128),
                             lambda h, t, *_, _o=roff // _RB: (t + _o, h)),
                pl.BlockSpec((T, 128), lambda h, t, *_: (0, hpairs + h)),
                pl.BlockSpec((T, 128), lambda h, t, *_: (0, 2 * hpairs + h)),
            ],
            out_specs=pl.BlockSpec((_RB, 128), lambda h, t, *_: (t, h)),
        )
        y_g = pl.pallas_call(
            functools.partial(_attn_group_body, w_g, roff),
            grid_spec=grid_spec,
            out_shape=jax.ShapeDtypeStruct((_GR, C), jnp.float32),
            compiler_params=pltpu.CompilerParams(
                dimension_semantics=("arbitrary", "arbitrary")),
        )(sparsity_ratios, qkv, qkv, qkv)
        y_groups.append(y_g)
    y2 = jnp.concatenate(y_groups, axis=0)
    out = pl.pallas_call(
        _proj_body,
        grid=(T // _RB,),
        in_specs=[
            pl.BlockSpec((_RB, C), lambda i: (i, 0)),
            pl.BlockSpec((C, C), lambda i: (0, 0)),
            pl.BlockSpec((1, C), lambda i: (0, 0)),
        ],
        out_specs=pl.BlockSpec((_RB, C), lambda i: (i, 0)),
        out_shape=jax.ShapeDtypeStruct((T, C), jnp.float32),
    )(y2, W_proj, b_proj.reshape(1, C))
    return out.reshape(B, T, C)


# speedup vs baseline: 4.1778x; 1.0295x over previous
import functools
import math

import jax
import jax.numpy as jnp
from jax.experimental import pallas as pl
from jax.experimental.pallas import tpu as pltpu

_T = 2048
_C = 768
_H = 12
_HD = _C // _H
_RB = 256
_GR = 512
_N_ITER = 12


def _qkv_body(x_ref, w_ref, b_ref, o_ref):
    o_ref[...] = jax.lax.dot_general(
        x_ref[...], w_ref[...], (((1,), (1,)), ((), ())),
        preferred_element_type=jnp.float32) + b_ref[...]


def _attn_body(ratio_ref, q_ref, k_ref, v_ref, o_ref):
    hp = pl.program_id(0)
    g = pl.program_id(1)
    tb = pl.program_id(2)
    for g_st in range(_T // _GR):

        @pl.when(g == g_st)
        def _():
            _attn_group(g_st * _GR + _GR, g_st * _GR, hp, tb,
                        ratio_ref, q_ref, k_ref, v_ref, o_ref)


def _attn_group(W, ROFF, hp, tb, ratio_ref, q_ref, k_ref, v_ref, o_ref):
    scale = 1.0 / math.sqrt(_HD)
    rows = ROFF + tb * _RB + jax.lax.broadcasted_iota(jnp.int32, (_RB, 1), 0)
    cols = jax.lax.broadcasted_iota(jnp.int32, (_RB, W), 1)
    valid = cols <= rows
    big = jnp.float32(3e38)
    tlen = (rows + 1).astype(jnp.float32)

    for sub in range(2):
        h = 2 * hp + sub
        q = q_ref[:, sub * _HD:(sub + 1) * _HD]
        k = k_ref[:W, sub * _HD:(sub + 1) * _HD]
        att = jax.lax.dot_general(
            q, k, (((1,), (1,)), ((), ())),
            preferred_element_type=jnp.float32) * scale

        att_m = jnp.where(valid, att, -big)
        mrow = jnp.max(att_m, axis=1, keepdims=True)
        lo = jnp.min(jnp.where(valid, att, big), axis=1, keepdims=True)

        r = ratio_ref[h]
        sig = 1.0 / (1.0 + jnp.exp(-r))
        kt = jnp.maximum(1, jnp.floor(tlen * sig).astype(jnp.int32))
        ktf = kt.astype(jnp.float32)

        def bisect(_, carry):
            lo_c, hi_c = carry
            mid = (lo_c + hi_c) * 0.5
            cnt = jnp.sum((att_m >= mid).astype(jnp.float32), axis=1,
                          keepdims=True)
            ge = cnt >= ktf
            return jnp.where(ge, mid, lo_c), jnp.where(ge, hi_c, mid)

        lo, _ = jax.lax.fori_loop(0, _N_ITER, bisect, (lo, mrow))

        m = jnp.maximum(mrow, 0.0)
        s = jnp.where(att_m >= lo, att_m, 0.0)
        p = jnp.exp(s - m)
        num = jax.lax.dot_general(
            p, v_ref[:W, sub * _HD:(sub + 1) * _HD], (((1,), (0,)), ((), ())),
            preferred_element_type=jnp.float32)
        den = jnp.sum(p, axis=1, keepdims=True)
        if W < _T:
            em = jnp.exp(-m)
            vtail = jnp.sum(v_ref[W:, sub * _HD:(sub + 1) * _HD], axis=0,
                            keepdims=True)
            num = num + em * vtail
            den = den + em * jnp.float32(_T - W)
        o_ref[:, sub * _HD:(sub + 1) * _HD] = num / den


def _proj_body(y_ref, w_ref, b_ref, o_ref):
    o_ref[...] = jax.lax.dot_general(
        y_ref[...], w_ref[...], (((1,), (1,)), ((), ())),
        preferred_element_type=jnp.float32) + b_ref[...]


@jax.jit
def kernel(x, W_attn, b_attn, W_proj, b_proj, sparsity_ratios):
    B, T, C = x.shape
    H = sparsity_ratios.shape[0]
    hd = C // H
    x2 = x.reshape(T, C)

    qkv = pl.pallas_call(
        _qkv_body,
        grid=(T // _RB,),
        in_specs=[
            pl.BlockSpec((_RB, C), lambda i: (i, 0)),
            pl.BlockSpec((3 * C, C), lambda i: (0, 0)),
            pl.BlockSpec((1, 3 * C), lambda i: (0, 0)),
        ],
        out_specs=pl.BlockSpec((_RB, 3 * C), lambda i: (i, 0)),
        out_shape=jax.ShapeDtypeStruct((T, 3 * C), jnp.float32),
    )(x2, W_attn, b_attn.reshape(1, 3 * C))

    hpairs = H // 2
    rpg = _GR // _RB
    grid_spec = pltpu.PrefetchScalarGridSpec(
        num_scalar_prefetch=1,
        grid=(hpairs, T // _GR, rpg),
        in_specs=[
            pl.BlockSpec((_RB, 128),
                         lambda h, g, t, *_: (g * rpg + t, h)),
            pl.BlockSpec((T, 128), lambda h, g, t, *_: (0, hpairs + h)),
            pl.BlockSpec((T, 128), lambda h, g, t, *_: (0, 2 * hpairs + h)),
        ],
        out_specs=pl.BlockSpec((_RB, 128),
                               lambda h, g, t, *_: (g * rpg + t, h)),
    )
    y2 = pl.pallas_call(
        _attn_body,
        grid_spec=grid_spec,
        out_shape=jax.ShapeDtypeStruct((T, C), jnp.float32),
        compiler_params=pltpu.CompilerParams(
            dimension_semantics=("arbitrary", "arbitrary", "arbitrary")),
    )(sparsity_ratios, qkv, qkv, qkv)
    out = pl.pallas_call(
        _proj_body,
        grid=(T // _RB,),
        in_specs=[
            pl.BlockSpec((_RB, C), lambda i: (i, 0)),
            pl.BlockSpec((C, C), lambda i: (0, 0)),
            pl.BlockSpec((1, C), lambda i: (0, 0)),
        ],
        out_specs=pl.BlockSpec((_RB, C), lambda i: (i, 0)),
        out_shape=jax.ShapeDtypeStruct((T, C), jnp.float32),
    )(y2, W_proj, b_proj.reshape(1, C))
    return out.reshape(B, T, C)


# speedup vs baseline: 4.9275x; 1.1795x over previous
import functools
import math

import jax
import jax.numpy as jnp
from jax.experimental import pallas as pl
from jax.experimental.pallas import tpu as pltpu

_T = 2048
_C = 768
_H = 12
_HD = _C // _H
_RB = 512
_GR = 512
_N_ITER = 12


def _qkv_body(x_ref, w_ref, b_ref, o_ref):
    o_ref[...] = jax.lax.dot_general(
        x_ref[...], w_ref[...], (((1,), (1,)), ((), ())),
        preferred_element_type=jnp.float32) + b_ref[...]


def _attn_body(ratio_ref, q_ref, k_ref, v_ref, o_ref):
    hp = pl.program_id(0)
    g = pl.program_id(1)
    tb = pl.program_id(2)
    for g_st in range(_T // _GR):

        @pl.when(g == g_st)
        def _():
            _attn_group(g_st * _GR + _GR, g_st * _GR, hp, tb,
                        ratio_ref, q_ref, k_ref, v_ref, o_ref)


def _attn_group(W, ROFF, hp, tb, ratio_ref, q_ref, k_ref, v_ref, o_ref):
    scale = 1.0 / math.sqrt(_HD)
    rows = ROFF + tb * _RB + jax.lax.broadcasted_iota(jnp.int32, (_RB, 1), 0)
    cols = jax.lax.broadcasted_iota(jnp.int32, (_RB, W), 1)
    valid = cols <= rows
    big = jnp.float32(3e38)
    tlen = (rows + 1).astype(jnp.float32)

    for sub in range(2):
        h = 2 * hp + sub
        q = q_ref[:, sub * _HD:(sub + 1) * _HD]
        k = k_ref[:W, sub * _HD:(sub + 1) * _HD]
        att = jax.lax.dot_general(
            q, k, (((1,), (1,)), ((), ())),
            preferred_element_type=jnp.float32) * scale

        att_m = jnp.where(valid, att, -big)
        mrow = jnp.max(att_m, axis=1, keepdims=True)
        lo = jnp.min(jnp.where(valid, att, big), axis=1, keepdims=True)

        r = ratio_ref[h]
        sig = 1.0 / (1.0 + jnp.exp(-r))
        kt = jnp.maximum(1, jnp.floor(tlen * sig).astype(jnp.int32))
        ktf = kt.astype(jnp.float32)

        def bisect(_, carry):
            lo_c, hi_c = carry
            mid = (lo_c + hi_c) * 0.5
            cnt = jnp.sum((att_m >= mid).astype(jnp.float32), axis=1,
                          keepdims=True)
            ge = cnt >= ktf
            return jnp.where(ge, mid, lo_c), jnp.where(ge, hi_c, mid)

        lo, _ = jax.lax.fori_loop(0, _N_ITER, bisect, (lo, mrow))

        m = jnp.maximum(mrow, 0.0)
        s = jnp.where(att_m >= lo, att_m, 0.0)
        p = jnp.exp(s - m)
        num = jax.lax.dot_general(
            p, v_ref[:W, sub * _HD:(sub + 1) * _HD], (((1,), (0,)), ((), ())),
            preferred_element_type=jnp.float32)
        den = jnp.sum(p, axis=1, keepdims=True)
        if W < _T:
            em = jnp.exp(-m)
            vtail = jnp.sum(v_ref[W:, sub * _HD:(sub + 1) * _HD], axis=0,
                            keepdims=True)
            num = num + em * vtail
            den = den + em * jnp.float32(_T - W)
        o_ref[:, sub * _HD:(sub + 1) * _HD] = num / den


def _proj_body(y_ref, w_ref, b_ref, o_ref):
    o_ref[...] = jax.lax.dot_general(
        y_ref[...], w_ref[...], (((1,), (1,)), ((), ())),
        preferred_element_type=jnp.float32) + b_ref[...]


@jax.jit
def kernel(x, W_attn, b_attn, W_proj, b_proj, sparsity_ratios):
    B, T, C = x.shape
    H = sparsity_ratios.shape[0]
    hd = C // H
    x2 = x.reshape(T, C)

    qkv = pl.pallas_call(
        _qkv_body,
        grid=(T // _RB,),
        in_specs=[
            pl.BlockSpec((_RB, C), lambda i: (i, 0)),
            pl.BlockSpec((3 * C, C), lambda i: (0, 0)),
            pl.BlockSpec((1, 3 * C), lambda i: (0, 0)),
        ],
        out_specs=pl.BlockSpec((_RB, 3 * C), lambda i: (i, 0)),
        out_shape=jax.ShapeDtypeStruct((T, 3 * C), jnp.float32),
    )(x2, W_attn, b_attn.reshape(1, 3 * C))

    hpairs = H // 2
    rpg = _GR // _RB
    grid_spec = pltpu.PrefetchScalarGridSpec(
        num_scalar_prefetch=1,
        grid=(hpairs, T // _GR, rpg),
        in_specs=[
            pl.BlockSpec((_RB, 128),
                         lambda h, g, t, *_: (g * rpg + t, h)),
            pl.BlockSpec((T, 128), lambda h, g, t, *_: (0, hpairs + h)),
            pl.BlockSpec((T, 128), lambda h, g, t, *_: (0, 2 * hpairs + h)),
        ],
        out_specs=pl.BlockSpec((_RB, 128),
                               lambda h, g, t, *_: (g * rpg + t, h)),
    )
    y2 = pl.pallas_call(
        _attn_body,
        grid_spec=grid_spec,
        out_shape=jax.ShapeDtypeStruct((T, C), jnp.float32),
        compiler_params=pltpu.CompilerParams(
            dimension_semantics=("arbitrary", "arbitrary", "arbitrary")),
    )(sparsity_ratios, qkv, qkv, qkv)
    out = pl.pallas_call(
        _proj_body,
        grid=(T // _RB,),
        in_specs=[
            pl.BlockSpec((_RB, C), lambda i: (i, 0)),
            pl.BlockSpec((C, C), lambda i: (0, 0)),
            pl.BlockSpec((1, C), lambda i: (0, 0)),
        ],
        out_specs=pl.BlockSpec((_RB, C), lambda i: (i, 0)),
        out_shape=jax.ShapeDtypeStruct((T, C), jnp.float32),
    )(y2, W_proj, b_proj.reshape(1, C))
    return out.reshape(B, T, C)


# speedup vs baseline: 5.5720x; 1.1308x over previous
import functools
import math

import jax
import jax.numpy as jnp
from jax.experimental import pallas as pl
from jax.experimental.pallas import tpu as pltpu

_T = 2048
_C = 768
_H = 12
_HD = _C // _H
_RB = 512
_GR = 512
_N_ITER = 10


def _qkv_body(x_ref, w_ref, b_ref, o_ref):
    o_ref[...] = jax.lax.dot_general(
        x_ref[...], w_ref[...], (((1,), (1,)), ((), ())),
        preferred_element_type=jnp.float32) + b_ref[...]


def _attn_body(ratio_ref, q_ref, k_ref, v_ref, o_ref):
    hp = pl.program_id(0)
    g = pl.program_id(1)
    tb = pl.program_id(2)
    for g_st in range(_T // _GR):

        @pl.when(g == g_st)
        def _():
            _attn_group(g_st * _GR + _GR, g_st * _GR, hp, tb,
                        ratio_ref, q_ref, k_ref, v_ref, o_ref)


def _attn_group(W, ROFF, hp, tb, ratio_ref, q_ref, k_ref, v_ref, o_ref):
    scale = 1.0 / math.sqrt(_HD)
    rows = ROFF + tb * _RB + jax.lax.broadcasted_iota(jnp.int32, (_RB, 1), 0)
    cols = jax.lax.broadcasted_iota(jnp.int32, (_RB, W), 1)
    valid = cols <= rows
    big = jnp.float32(3e38)
    tlen = (rows + 1).astype(jnp.float32)

    for sub in range(2):
        h = 2 * hp + sub
        q = q_ref[:, sub * _HD:(sub + 1) * _HD]
        k = k_ref[:W, sub * _HD:(sub + 1) * _HD]
        att = jax.lax.dot_general(
            q, k, (((1,), (1,)), ((), ())),
            preferred_element_type=jnp.float32) * scale

        att_m = jnp.where(valid, att, -big)
        mrow = jnp.max(att_m, axis=1, keepdims=True)
        lo = jnp.min(jnp.where(valid, att, big), axis=1, keepdims=True)

        r = ratio_ref[h]
        sig = 1.0 / (1.0 + jnp.exp(-r))
        kt = jnp.maximum(1, jnp.floor(tlen * sig).astype(jnp.int32))
        ktf = kt.astype(jnp.float32)

        def bisect(_, carry):
            lo_c, hi_c = carry
            mid = (lo_c + hi_c) * 0.5
            cnt = jnp.sum((att_m >= mid).astype(jnp.float32), axis=1,
                          keepdims=True)
            ge = cnt >= ktf
            return jnp.where(ge, mid, lo_c), jnp.where(ge, hi_c, mid)

        lo, _ = jax.lax.fori_loop(0, _N_ITER, bisect, (lo, mrow))

        m = jnp.maximum(mrow, 0.0)
        s = jnp.where(att_m >= lo, att_m, 0.0)
        p = jnp.exp(s - m)
        num = jax.lax.dot_general(
            p, v_ref[:W, sub * _HD:(sub + 1) * _HD], (((1,), (0,)), ((), ())),
            preferred_element_type=jnp.float32)
        den = jnp.sum(p, axis=1, keepdims=True)
        if W < _T:
            em = jnp.exp(-m)
            vtail = jnp.sum(v_ref[W:, sub * _HD:(sub + 1) * _HD], axis=0,
                            keepdims=True)
            num = num + em * vtail
            den = den + em * jnp.float32(_T - W)
        o_ref[:, sub * _HD:(sub + 1) * _HD] = num / den


def _proj_body(y_ref, w_ref, b_ref, o_ref):
    o_ref[...] = jax.lax.dot_general(
        y_ref[...], w_ref[...], (((1,), (1,)), ((), ())),
        preferred_element_type=jnp.float32) + b_ref[...]


@jax.jit
def kernel(x, W_attn, b_attn, W_proj, b_proj, sparsity_ratios):
    B, T, C = x.shape
    H = sparsity_ratios.shape[0]
    hd = C // H
    x2 = x.reshape(T, C)

    qkv = pl.pallas_call(
        _qkv_body,
        grid=(T // _RB,),
        in_specs=[
            pl.BlockSpec((_RB, C), lambda i: (i, 0)),
            pl.BlockSpec((3 * C, C), lambda i: (0, 0)),
            pl.BlockSpec((1, 3 * C), lambda i: (0, 0)),
        ],
        out_specs=pl.BlockSpec((_RB, 3 * C), lambda i: (i, 0)),
        out_shape=jax.ShapeDtypeStruct((T, 3 * C), jnp.float32),
    )(x2, W_attn, b_attn.reshape(1, 3 * C))

    hpairs = H // 2
    rpg = _GR // _RB
    grid_spec = pltpu.PrefetchScalarGridSpec(
        num_scalar_prefetch=1,
        grid=(hpairs, T // _GR, rpg),
        in_specs=[
            pl.BlockSpec((_RB, 128),
                         lambda h, g, t, *_: (g * rpg + t, h)),
            pl.BlockSpec((T, 128), lambda h, g, t, *_: (0, hpairs + h)),
            pl.BlockSpec((T, 128), lambda h, g, t, *_: (0, 2 * hpairs + h)),
        ],
        out_specs=pl.BlockSpec((_RB, 128),
                               lambda h, g, t, *_: (g * rpg + t, h)),
    )
    y2 = pl.pallas_call(
        _attn_body,
        grid_spec=grid_spec,
        out_shape=jax.ShapeDtypeStruct((T, C), jnp.float32),
        compiler_params=pltpu.CompilerParams(
            dimension_semantics=("arbitrary", "arbitrary", "arbitrary")),
    )(sparsity_ratios, qkv, qkv, qkv)
    out = pl.pallas_call(
        _proj_body,
        grid=(T // _RB,),
        in_specs=[
            pl.BlockSpec((_RB, C), lambda i: (i, 0)),
            pl.BlockSpec((C, C), lambda i: (0, 0)),
            pl.BlockSpec((1, C), lambda i: (0, 0)),
        ],
        out_specs=pl.BlockSpec((_RB, C), lambda i: (i, 0)),
        out_shape=jax.ShapeDtypeStruct((T, C), jnp.float32),
    )(y2, W_proj, b_proj.reshape(1, C))
    return out.reshape(B, T, C)


# speedup vs baseline: 6.7104x; 1.2043x over previous
import functools
import math

import jax
import jax.numpy as jnp
from jax.experimental import pallas as pl
from jax.experimental.pallas import tpu as pltpu

_T = 2048
_C = 768
_H = 12
_HD = _C // _H
_RB = 512
_GR = 512
_N_ITER = 10


def _qkv_body(x_ref, w_ref, b_ref, o_ref):
    o_ref[...] = jax.lax.dot_general(
        x_ref[...], w_ref[...], (((1,), (1,)), ((), ())),
        preferred_element_type=jnp.float32) + b_ref[...]


def _attn_body(ratio_ref, q_ref, k_ref, v_ref, o_ref):
    hp = pl.program_id(0)
    g = pl.program_id(1)
    tb = pl.program_id(2)
    for g_st in range(_T // _GR):

        @pl.when(g == g_st)
        def _():
            _attn_group(g_st * _GR + _GR, g_st * _GR, hp, tb,
                        ratio_ref, q_ref, k_ref, v_ref, o_ref)


def _attn_group(W, ROFF, hp, tb, ratio_ref, q_ref, k_ref, v_ref, o_ref):
    scale = 1.0 / math.sqrt(_HD)
    rows = ROFF + tb * _RB + jax.lax.broadcasted_iota(jnp.int32, (_RB, 1), 0)
    cols = jax.lax.broadcasted_iota(jnp.int32, (_RB, W), 1)
    valid = cols <= rows
    big = jnp.float32(3e38)
    tlen = (rows + 1).astype(jnp.float32)

    for sub in range(2):
        h = 2 * hp + sub
        q = q_ref[:, sub * _HD:(sub + 1) * _HD]
        k = k_ref[:W, sub * _HD:(sub + 1) * _HD]
        att = jax.lax.dot_general(
            q, k, (((1,), (1,)), ((), ())),
            preferred_element_type=jnp.float32) * scale

        att_m = jnp.where(valid, att, -big)
        mrow = jnp.max(att_m, axis=1, keepdims=True)
        lo = jnp.min(jnp.where(valid, att, big), axis=1, keepdims=True)

        r = ratio_ref[h]
        sig = 1.0 / (1.0 + jnp.exp(-r))
        kt = jnp.maximum(1, jnp.floor(tlen * sig).astype(jnp.int32))
        ktf = kt.astype(jnp.float32)

        hi = mrow
        for _ in range(_N_ITER):
            mid = (lo + hi) * 0.5
            cnt = jnp.sum((att_m >= mid).astype(jnp.float32), axis=1,
                          keepdims=True)
            ge = cnt >= ktf
            lo = jnp.where(ge, mid, lo)
            hi = jnp.where(ge, hi, mid)

        m = jnp.maximum(mrow, 0.0)
        s = jnp.where(att_m >= lo, att_m, 0.0)
        p = jnp.exp(s - m)
        num = jax.lax.dot_general(
            p, v_ref[:W, sub * _HD:(sub + 1) * _HD], (((1,), (0,)), ((), ())),
            preferred_element_type=jnp.float32)
        den = jnp.sum(p, axis=1, keepdims=True)
        if W < _T:
            em = jnp.exp(-m)
            vtail = jnp.sum(v_ref[W:, sub * _HD:(sub + 1) * _HD], axis=0,
                            keepdims=True)
            num = num + em * vtail
            den = den + em * jnp.float32(_T - W)
        o_ref[:, sub * _HD:(sub + 1) * _HD] = num / den


def _proj_body(y_ref, w_ref, b_ref, o_ref):
    o_ref[...] = jax.lax.dot_general(
        y_ref[...], w_ref[...], (((1,), (1,)), ((), ())),
        preferred_element_type=jnp.float32) + b_ref[...]


@jax.jit
def kernel(x, W_attn, b_attn, W_proj, b_proj, sparsity_ratios):
    B, T, C = x.shape
    H = sparsity_ratios.shape[0]
    hd = C // H
    x2 = x.reshape(T, C)

    qkv = pl.pallas_call(
        _qkv_body,
        grid=(T // _RB,),
        in_specs=[
            pl.BlockSpec((_RB, C), lambda i: (i, 0)),
            pl.BlockSpec((3 * C, C), lambda i: (0, 0)),
            pl.BlockSpec((1, 3 * C), lambda i: (0, 0)),
        ],
        out_specs=pl.BlockSpec((_RB, 3 * C), lambda i: (i, 0)),
        out_shape=jax.ShapeDtypeStruct((T, 3 * C), jnp.float32),
    )(x2, W_attn, b_attn.reshape(1, 3 * C))

    hpairs = H // 2
    rpg = _GR // _RB
    grid_spec = pltpu.PrefetchScalarGridSpec(
        num_scalar_prefetch=1,
        grid=(hpairs, T // _GR, rpg),
        in_specs=[
            pl.BlockSpec((_RB, 128),
                         lambda h, g, t, *_: (g * rpg + t, h)),
            pl.BlockSpec((T, 128), lambda h, g, t, *_: (0, hpairs + h)),
            pl.BlockSpec((T, 128), lambda h, g, t, *_: (0, 2 * hpairs + h)),
        ],
        out_specs=pl.BlockSpec((_RB, 128),
                               lambda h, g, t, *_: (g * rpg + t, h)),
    )
    y2 = pl.pallas_call(
        _attn_body,
        grid_spec=grid_spec,
        out_shape=jax.ShapeDtypeStruct((T, C), jnp.float32),
        compiler_params=pltpu.CompilerParams(
            dimension_semantics=("arbitrary", "arbitrary", "arbitrary")),
    )(sparsity_ratios, qkv, qkv, qkv)
    out = pl.pallas_call(
        _proj_body,
        grid=(T // _RB,),
        in_specs=[
            pl.BlockSpec((_RB, C), lambda i: (i, 0)),
            pl.BlockSpec((C, C), lambda i: (0, 0)),
            pl.BlockSpec((1, C), lambda i: (0, 0)),
        ],
        out_specs=pl.BlockSpec((_RB, C), lambda i: (i, 0)),
        out_shape=jax.ShapeDtypeStruct((T, C), jnp.float32),
    )(y2, W_proj, b_proj.reshape(1, C))
    return out.reshape(B, T, C)


# speedup vs baseline: 6.9694x; 1.0386x over previous
import functools
import math

import jax
import jax.numpy as jnp
from jax.experimental import pallas as pl
from jax.experimental.pallas import tpu as pltpu

_T = 2048
_C = 768
_H = 12
_HD = _C // _H
_RB = 512
_GR = 512
_N_ITER = 10


def _qkv_body(x_ref, w_ref, b_ref, o_ref):
    o_ref[...] = jax.lax.dot_general(
        x_ref[...], w_ref[...], (((1,), (1,)), ((), ())),
        preferred_element_type=jnp.float32) + b_ref[...]


def _attn_body(ratio_ref, q_ref, k_ref, v_ref, o_ref):
    hp = pl.program_id(0)
    g = pl.program_id(1)
    tb = pl.program_id(2)
    for g_st in range(_T // _GR):

        @pl.when(g == g_st)
        def _():
            _attn_group(g_st * _GR + _GR, g_st * _GR, hp, tb,
                        ratio_ref, q_ref, k_ref, v_ref, o_ref)


def _attn_group(W, ROFF, hp, tb, ratio_ref, q_ref, k_ref, v_ref, o_ref):
    scale = 1.0 / math.sqrt(_HD)
    rows = ROFF + tb * _RB + jax.lax.broadcasted_iota(jnp.int32, (_RB, 1), 0)
    cols = jax.lax.broadcasted_iota(jnp.int32, (_RB, W), 1)
    valid = cols <= rows
    big = jnp.float32(3e38)
    tlen = (rows + 1).astype(jnp.float32)

    att_m, mrow, lo, hi, ktf = [None, None], [None, None], [None, None], \
        [None, None], [None, None]
    for sub in range(2):
        q = q_ref[:, sub * _HD:(sub + 1) * _HD]
        k = k_ref[:W, sub * _HD:(sub + 1) * _HD]
        att = jax.lax.dot_general(
            q, k, (((1,), (1,)), ((), ())),
            preferred_element_type=jnp.float32) * scale
        att_m[sub] = jnp.where(valid, att, -big)
        mrow[sub] = jnp.max(att_m[sub], axis=1, keepdims=True)
        lo[sub] = jnp.min(jnp.where(valid, att, big), axis=1, keepdims=True)
        hi[sub] = mrow[sub]
        r = ratio_ref[2 * hp + sub]
        sig = 1.0 / (1.0 + jnp.exp(-r))
        kt = jnp.maximum(1, jnp.floor(tlen * sig).astype(jnp.int32))
        ktf[sub] = kt.astype(jnp.float32)

    for _ in range(_N_ITER):
        for sub in range(2):
            mid = (lo[sub] + hi[sub]) * 0.5
            cnt = jnp.sum((att_m[sub] >= mid).astype(jnp.float32), axis=1,
                          keepdims=True)
            ge = cnt >= ktf[sub]
            lo[sub] = jnp.where(ge, mid, lo[sub])
            hi[sub] = jnp.where(ge, hi[sub], mid)

    for sub in range(2):
        m = jnp.maximum(mrow[sub], 0.0)
        s = jnp.where(att_m[sub] >= lo[sub], att_m[sub], 0.0)
        p = jnp.exp(s - m)
        num = jax.lax.dot_general(
            p, v_ref[:W, sub * _HD:(sub + 1) * _HD], (((1,), (0,)), ((), ())),
            preferred_element_type=jnp.float32)
        den = jnp.sum(p, axis=1, keepdims=True)
        if W < _T:
            em = jnp.exp(-m)
            vtail = jnp.sum(v_ref[W:, sub * _HD:(sub + 1) * _HD], axis=0,
                            keepdims=True)
            num = num + em * vtail
            den = den + em * jnp.float32(_T - W)
        o_ref[:, sub * _HD:(sub + 1) * _HD] = num / den


def _proj_body(y_ref, w_ref, b_ref, o_ref):
    o_ref[...] = jax.lax.dot_general(
        y_ref[...], w_ref[...], (((1,), (1,)), ((), ())),
        preferred_element_type=jnp.float32) + b_ref[...]


@jax.jit
def kernel(x, W_attn, b_attn, W_proj, b_proj, sparsity_ratios):
    B, T, C = x.shape
    H = sparsity_ratios.shape[0]
    hd = C // H
    x2 = x.reshape(T, C)

    qkv = pl.pallas_call(
        _qkv_body,
        grid=(T // _RB,),
        in_specs=[
            pl.BlockSpec((_RB, C), lambda i: (i, 0)),
            pl.BlockSpec((3 * C, C), lambda i: (0, 0)),
            pl.BlockSpec((1, 3 * C), lambda i: (0, 0)),
        ],
        out_specs=pl.BlockSpec((_RB, 3 * C), lambda i: (i, 0)),
        out_shape=jax.ShapeDtypeStruct((T, 3 * C), jnp.float32),
    )(x2, W_attn, b_attn.reshape(1, 3 * C))

    hpairs = H // 2
    rpg = _GR // _RB
    grid_spec = pltpu.PrefetchScalarGridSpec(
        num_scalar_prefetch=1,
        grid=(hpairs, T // _GR, rpg),
        in_specs=[
            pl.BlockSpec((_RB, 128),
                         lambda h, g, t, *_: (g * rpg + t, h)),
            pl.BlockSpec((T, 128), lambda h, g, t, *_: (0, hpairs + h)),
            pl.BlockSpec((T, 128), lambda h, g, t, *_: (0, 2 * hpairs + h)),
        ],
        out_specs=pl.BlockSpec((_RB, 128),
                               lambda h, g, t, *_: (g * rpg + t, h)),
    )
    y2 = pl.pallas_call(
        _attn_body,
        grid_spec=grid_spec,
        out_shape=jax.ShapeDtypeStruct((T, C), jnp.float32),
        compiler_params=pltpu.CompilerParams(
            dimension_semantics=("arbitrary", "arbitrary", "arbitrary")),
    )(sparsity_ratios, qkv, qkv, qkv)
    out = pl.pallas_call(
        _proj_body,
        grid=(T // _RB,),
        in_specs=[
            pl.BlockSpec((_RB, C), lambda i: (i, 0)),
            pl.BlockSpec((C, C), lambda i: (0, 0)),
            pl.BlockSpec((1, C), lambda i: (0, 0)),
        ],
        out_specs=pl.BlockSpec((_RB, C), lambda i: (i, 0)),
        out_shape=jax.ShapeDtypeStruct((T, C), jnp.float32),
    )(y2, W_proj, b_proj.reshape(1, C))
    return out.reshape(B, T, C)
